# baseline (speedup 1.0000x reference)
import functools

import jax
import jax.numpy as jnp
from jax import lax
from jax.experimental import pallas as pl
from jax.experimental.pallas import tpu as pltpu
from jax.experimental.pallas import tpu_sc as plsc

_B = 1024
_K = 1024
_D = 64
_BB = 1024
_DP = 128


def _vq_body(s_ref, l_ref, w0_ref, b0_ref, w1_ref, b1_ref, w2_ref, b2_ref,
             cb_ref, u_ref, idx_ref):
    x = jnp.concatenate((s_ref[...], l_ref[...]), axis=1)
    h = jnp.maximum(
        jnp.dot(x, w0_ref[...], preferred_element_type=jnp.float32) + b0_ref[...], 0.0)
    h = jnp.maximum(
        jnp.dot(h, w1_ref[...], preferred_element_type=jnp.float32) + b1_ref[...], 0.0)
    u = jnp.dot(h, w2_ref[...], preferred_element_type=jnp.float32) + b2_ref[...]
    u_ref[...] = u

    u_aug = jnp.concatenate((u * -2.0, jnp.ones((_BB, 1), jnp.float32)), axis=1)
    cb = cb_ref[...]
    c2 = jnp.sum(cb * cb, axis=1, keepdims=True)
    cb_aug = jnp.concatenate((cb, c2), axis=1)
    d = lax.dot_general(u_aug, cb_aug, (((1,), (1,)), ((), ())),
                        preferred_element_type=jnp.float32,
                        precision=lax.Precision.HIGHEST)
    dmin = jnp.min(d, axis=1, keepdims=True)
    iota = lax.broadcasted_iota(jnp.int32, d.shape, 1)
    idx_ref[...] = jnp.min(jnp.where(d == dmin, iota, _K), axis=1,
                           keepdims=True)


def _tc_stage(skills, language_operators, W0, b0, W1, b1, W2, b2, codebook):
    nsteps = _B // _BB
    row_block = lambda i: (i, 0)
    whole = lambda i: (0, 0)
    return pl.pallas_call(
        _vq_body,
        grid=(nsteps,),
        in_specs=[
            pl.BlockSpec((_BB, 128), row_block),
            pl.BlockSpec((_BB, 512), row_block),
            pl.BlockSpec((640, 256), whole),
            pl.BlockSpec((1, 256), whole),
            pl.BlockSpec((256, 256), whole),
            pl.BlockSpec((1, 256), whole),
            pl.BlockSpec((256, _D), whole),
            pl.BlockSpec((1, _D), whole),
            pl.BlockSpec((_K, _D), whole),
        ],
        out_specs=(
            pl.BlockSpec((_BB, _D), row_block),
            pl.BlockSpec((_BB, 1), row_block),
        ),
        out_shape=(
            jax.ShapeDtypeStruct((_B, _D), jnp.float32),
            jax.ShapeDtypeStruct((_B, 1), jnp.int32),
        ),
        compiler_params=pltpu.CompilerParams(
            dimension_semantics=("arbitrary",),
        ),
    )(skills, language_operators, W0, b0[None, :], W1, b1[None, :],
      W2, b2[None, :], codebook)


def _sc_gather(codebook_padded, idx):
    nb = idx.shape[0]
    info = plsc.get_sparse_core_info()
    nc, ns = info.num_cores, info.num_subcores
    nw = nc * ns
    b_per_w = nb // nw
    mesh = plsc.VectorSubcoreMesh(core_axis_name="c", subcore_axis_name="s")

    @functools.partial(
        pl.kernel, mesh=mesh,
        out_type=jax.ShapeDtypeStruct((nb, _DP), jnp.float32),
        scratch_types=[
            pltpu.VMEM((b_per_w,), jnp.int32),
            pltpu.VMEM((b_per_w, _DP), jnp.float32),
            pltpu.SemaphoreType.DMA,
        ],
    )
    def gather_kernel(cb_hbm, idx_hbm, out_hbm, idx_v, rows_v, sem):
        wid = lax.axis_index("s") * nc + lax.axis_index("c")
        base = wid * b_per_w
        pltpu.sync_copy(idx_hbm.at[pl.ds(base, b_per_w)], idx_v)
        pltpu.async_copy(cb_hbm.at[idx_v], rows_v, sem).wait()
        pltpu.sync_copy(rows_v, out_hbm.at[pl.ds(base, b_per_w)])

    return gather_kernel(codebook_padded, idx)


def kernel(skills, language_operators, W0, b0, W1, b1, W2, b2, codebook):
    u, idx2d = _tc_stage(skills, language_operators, W0, b0, W1, b1, W2, b2,
                         codebook)
    cb_pad = jnp.pad(codebook, ((0, 0), (0, _DP - _D)))
    idx = idx2d.reshape(_B)
    q0 = _sc_gather(cb_pad, idx[:_B // 2])
    q1 = _sc_gather(cb_pad, idx[_B // 2:])
    q = jnp.concatenate((q0, q1), axis=0)[:, :_D]
    return (u, q)


# speedup vs baseline: 1.0781x; 1.0781x over previous
import functools

import jax
import jax.numpy as jnp
from jax import lax
from jax.experimental import pallas as pl
from jax.experimental.pallas import tpu as pltpu
from jax.experimental.pallas import tpu_sc as plsc

_B = 1024
_K = 1024
_D = 64
_BB = 1024
_DP = 128


def _vq_body(s_ref, l_ref, w0_ref, b0_ref, w1_ref, b1_ref, w2_ref, b2_ref,
             cb_ref, u_ref, idx_ref):
    x = jnp.concatenate((s_ref[...], l_ref[...]), axis=1)
    h = jnp.maximum(
        jnp.dot(x, w0_ref[...], preferred_element_type=jnp.float32) + b0_ref[...], 0.0)
    h = jnp.maximum(
        jnp.dot(h, w1_ref[...], preferred_element_type=jnp.float32) + b1_ref[...], 0.0)
    u = jnp.dot(h, w2_ref[...], preferred_element_type=jnp.float32) + b2_ref[...]
    u_ref[...] = u

    u_aug = jnp.concatenate((u * -2.0, jnp.ones((_BB, 1), jnp.float32)), axis=1)
    cb = cb_ref[...]
    c2 = jnp.sum(cb * cb, axis=1, keepdims=True)
    cb_aug = jnp.concatenate((cb, c2), axis=1)
    d = lax.dot_general(u_aug, cb_aug, (((1,), (1,)), ((), ())),
                        preferred_element_type=jnp.float32,
                        precision=lax.Precision.HIGHEST)
    dmin = jnp.min(d, axis=1, keepdims=True)
    iota = lax.broadcasted_iota(jnp.int32, d.shape, 1)
    idx_ref[...] = jnp.min(jnp.where(d == dmin, iota, _K), axis=1,
                           keepdims=True)


def _tc_stage(skills, language_operators, W0, b0, W1, b1, W2, b2, codebook):
    nsteps = _B // _BB
    row_block = lambda i: (i, 0)
    whole = lambda i: (0, 0)
    return pl.pallas_call(
        _vq_body,
        grid=(nsteps,),
        in_specs=[
            pl.BlockSpec((_BB, 128), row_block),
            pl.BlockSpec((_BB, 512), row_block),
            pl.BlockSpec((640, 256), whole),
            pl.BlockSpec((1, 256), whole),
            pl.BlockSpec((256, 256), whole),
            pl.BlockSpec((1, 256), whole),
            pl.BlockSpec((256, _D), whole),
            pl.BlockSpec((1, _D), whole),
            pl.BlockSpec((_K, _D), whole),
        ],
        out_specs=(
            pl.BlockSpec((_BB, _D), row_block),
            pl.BlockSpec((_BB, 1), row_block),
        ),
        out_shape=(
            jax.ShapeDtypeStruct((_B, _D), jnp.float32),
            jax.ShapeDtypeStruct((_B, 1), jnp.int32),
        ),
        compiler_params=pltpu.CompilerParams(
            dimension_semantics=("arbitrary",),
        ),
    )(skills, language_operators, W0, b0[None, :], W1, b1[None, :],
      W2, b2[None, :], codebook)


def _sc_row_gather(src, idx):
    n_out = idx.shape[0]
    info = plsc.get_sparse_core_info()
    nc, ns = info.num_cores, info.num_subcores
    nw = nc * ns
    b_per_w = n_out // nw
    mesh = plsc.VectorSubcoreMesh(core_axis_name="c", subcore_axis_name="s")

    @functools.partial(
        pl.kernel, mesh=mesh,
        out_type=jax.ShapeDtypeStruct((n_out, _DP), jnp.float32),
        scratch_types=[
            pltpu.VMEM((b_per_w,), jnp.int32),
            pltpu.VMEM((b_per_w, _DP), jnp.float32),
            pltpu.SemaphoreType.DMA,
        ],
    )
    def gather_kernel(src_hbm, idx_hbm, out_hbm, idx_v, rows_v, sem):
        wid = lax.axis_index("s") * nc + lax.axis_index("c")
        base = wid * b_per_w
        pltpu.sync_copy(idx_hbm.at[pl.ds(base, b_per_w)], idx_v)
        pltpu.async_copy(src_hbm.at[idx_v], rows_v, sem).wait()
        pltpu.sync_copy(rows_v, out_hbm.at[pl.ds(base, b_per_w)])

    return gather_kernel(src, idx)


def kernel(skills, language_operators, W0, b0, W1, b1, W2, b2, codebook):
    cb128 = codebook.reshape(_K // 2, _DP)
    cb_staged = _sc_row_gather(cb128, jnp.arange(_K // 2, dtype=jnp.int32))

    u, idx2d = _tc_stage(skills, language_operators, W0, b0, W1, b1, W2, b2,
                         codebook)
    idx = idx2d.reshape(_B)

    wide = _sc_row_gather(cb_staged, idx >> 1)
    q = jnp.where((idx & 1)[:, None] == 1, wide[:, _D:], wide[:, :_D])
    return (u, q)


# speedup vs baseline: 1.0951x; 1.0157x over previous
import functools

import jax
import jax.numpy as jnp
from jax import lax
from jax.experimental import pallas as pl
from jax.experimental.pallas import tpu as pltpu
from jax.experimental.pallas import tpu_sc as plsc

_B = 1024
_K = 1024
_D = 64
_BB = 1024
_DP = 128


def _vq_body(s_ref, l_ref, w0_ref, b0_ref, w1_ref, b1_ref, w2_ref, b2_ref,
             cb_ref, u_ref, idx_ref):
    x = jnp.concatenate((s_ref[...], l_ref[...]), axis=1)
    h = jnp.maximum(
        jnp.dot(x, w0_ref[...], preferred_element_type=jnp.float32) + b0_ref[...], 0.0)
    h = jnp.maximum(
        jnp.dot(h, w1_ref[...], preferred_element_type=jnp.float32) + b1_ref[...], 0.0)
    u = jnp.dot(h, w2_ref[...], preferred_element_type=jnp.float32) + b2_ref[...]
    u_ref[...] = u

    u_aug = jnp.concatenate((u * -2.0, jnp.ones((_BB, 1), jnp.float32)), axis=1)
    cb = cb_ref[...]
    c2 = jnp.sum(cb * cb, axis=1, keepdims=True)
    cb_aug = jnp.concatenate((cb, c2), axis=1)
    d = lax.dot_general(u_aug, cb_aug, (((1,), (1,)), ((), ())),
                        preferred_element_type=jnp.float32,
                        precision=lax.Precision.HIGHEST)
    dmin = jnp.min(d, axis=1, keepdims=True)
    iota = lax.broadcasted_iota(jnp.int32, d.shape, 1)
    idx_ref[...] = jnp.min(jnp.where(d == dmin, iota, _K), axis=1,
                           keepdims=True)


def _tc_stage(skills, language_operators, W0, b0, W1, b1, W2, b2, codebook):
    nsteps = _B // _BB
    row_block = lambda i: (i, 0)
    whole = lambda i: (0, 0)
    return pl.pallas_call(
        _vq_body,
        grid=(nsteps,),
        in_specs=[
            pl.BlockSpec((_BB, 128), row_block),
            pl.BlockSpec((_BB, 512), row_block),
            pl.BlockSpec((640, 256), whole),
            pl.BlockSpec((1, 256), whole),
            pl.BlockSpec((256, 256), whole),
            pl.BlockSpec((1, 256), whole),
            pl.BlockSpec((256, _D), whole),
            pl.BlockSpec((1, _D), whole),
            pl.BlockSpec((_K, _D), whole),
        ],
        out_specs=(
            pl.BlockSpec((_BB, _D), row_block),
            pl.BlockSpec((_BB, 1), row_block),
        ),
        out_shape=(
            jax.ShapeDtypeStruct((_B, _D), jnp.float32),
            jax.ShapeDtypeStruct((_B, 1), jnp.int32),
        ),
        compiler_params=pltpu.CompilerParams(
            dimension_semantics=("arbitrary",),
        ),
    )(skills, language_operators, W0, b0[None, :], W1, b1[None, :],
      W2, b2[None, :], codebook)


def _sc_gather(codebook_padded, idx):
    info = plsc.get_sparse_core_info()
    nc, ns = info.num_cores, info.num_subcores
    nw = nc * ns
    b_per_w = _B // nw
    mesh = plsc.VectorSubcoreMesh(core_axis_name="c", subcore_axis_name="s")

    @functools.partial(
        pl.kernel, mesh=mesh,
        out_type=jax.ShapeDtypeStruct((_B, _DP), jnp.float32),
        scratch_types=[
            pltpu.VMEM((b_per_w,), jnp.int32),
            pltpu.VMEM((b_per_w, _DP), jnp.float32),
            pltpu.SemaphoreType.DMA,
        ],
    )
    def gather_kernel(cb_hbm, idx_hbm, out_hbm, idx_v, rows_v, sem):
        wid = lax.axis_index("s") * nc + lax.axis_index("c")
        base = wid * b_per_w
        pltpu.sync_copy(idx_hbm.at[pl.ds(base, b_per_w)], idx_v)
        pltpu.async_copy(cb_hbm.at[idx_v], rows_v, sem).wait()
        pltpu.sync_copy(rows_v, out_hbm.at[pl.ds(base, b_per_w)])

    return gather_kernel(codebook_padded, idx)


def kernel(skills, language_operators, W0, b0, W1, b1, W2, b2, codebook):
    u, idx2d = _tc_stage(skills, language_operators, W0, b0, W1, b1, W2, b2,
                         codebook)
    cb_pad = jnp.pad(codebook, ((0, 0), (0, _DP - _D)))
    q = _sc_gather(cb_pad, idx2d.reshape(_B))[:, :_D]
    return (u, q)
